# baseline (speedup 1.0000x reference)
import jax
import jax.numpy as jnp
from jax.experimental import pallas as pl

_BS, _NA, _FH, _FW = 64, 9, 50, 84
_BB = 8


def _p_body(in_ref, out_ref):
    x = in_ref[...]
    y = x.reshape(2, _NA, _FH, _BB, _FW)
    y = y.transpose(3, 1, 2, 0, 4)
    out_ref[...] = y.reshape(_BB * _NA * 2 * _FH, _FW)


def _r_body(in_ref, out_ref):
    x = in_ref[...]
    y = x.reshape(4, _NA, 2, _BB, _FW)
    y = y.transpose(3, 1, 2, 0, 4)
    out_ref[...] = y.reshape(_BB, _NA, 8, _FW)


def kernel(preds, regs):
    pin = jnp.transpose(preds, (1, 2, 0, 3))
    rin = jnp.transpose(regs, (1, 2, 0, 3))

    p4 = pl.pallas_call(
        _p_body,
        grid=(_BS // _BB,),
        in_specs=[pl.BlockSpec((2 * _NA, _FH, _BB, _FW),
                               lambda bt: (0, 0, bt, 0))],
        out_specs=pl.BlockSpec((_BB * _NA * 2 * _FH, _FW),
                               lambda bt: (bt, 0)),
        out_shape=jax.ShapeDtypeStruct((_BS * _NA * 2 * _FH, _FW),
                                       jnp.float32),
    )(pin)

    r4 = pl.pallas_call(
        _r_body,
        grid=(_BS // _BB, _FH // 2),
        in_specs=[pl.BlockSpec((4 * _NA, 2, _BB, _FW),
                               lambda bt, ht: (0, ht, bt, 0))],
        out_specs=pl.BlockSpec((_BB, _NA, 8, _FW),
                               lambda bt, ht: (bt, 0, ht, 0)),
        out_shape=jax.ShapeDtypeStruct((_BS, _NA, 4 * _FH, _FW), jnp.float32),
    )(rin)

    p5 = p4.reshape(_BS, _NA, _FH, 2, _FW)
    r5 = r4.reshape(_BS, _NA, _FH, 4, _FW)
    return (jnp.swapaxes(p5, 3, 4), jnp.swapaxes(r5, 3, 4))


# speedup vs baseline: 1.9693x; 1.9693x over previous
import jax
import jax.numpy as jnp
from jax.experimental import pallas as pl

_BS, _NA, _FH, _FW = 64, 9, 50, 84


def _make_body(nc):
    def body(*refs):
        out_ref = refs[-1]
        for c in range(nc):
            out_ref[:, 0, :, c, :] = jnp.swapaxes(refs[c][0], 0, 1)
    return body


def _permute(x, nc):
    return pl.pallas_call(
        _make_body(nc),
        grid=(_NA,),
        in_specs=[
            pl.BlockSpec((1, _FH, _BS, _FW),
                         lambda a, c=c: (c * _NA + a, 0, 0, 0))
            for c in range(nc)
        ],
        out_specs=pl.BlockSpec(
            (_BS, 1, _FH, nc, _FW), lambda a: (0, a, 0, 0, 0)),
        out_shape=jax.ShapeDtypeStruct((_BS, _NA, _FH, nc, _FW), jnp.float32),
    )(*([x] * nc))


def kernel(preds, regs):
    pin = jnp.transpose(preds, (1, 2, 0, 3))
    rin = jnp.transpose(regs, (1, 2, 0, 3))
    p5 = _permute(pin, 2)
    r5 = _permute(rin, 4)
    return (
        jnp.swapaxes(p5, 3, 4),
        jnp.swapaxes(r5, 3, 4),
    )


# speedup vs baseline: 2.0687x; 1.0505x over previous
import jax
import jax.numpy as jnp
from jax.experimental import pallas as pl

_BS, _NA, _FH, _FW = 64, 9, 50, 84
_HH = _FH // 2


def _body(p0, p1, r0, r1, r2, r3, po, ro):
    for c, ref in enumerate((p0, p1)):
        po[:, 0, :, c, :] = jnp.swapaxes(ref[0], 0, 1)
    for c, ref in enumerate((r0, r1, r2, r3)):
        ro[:, 0, :, c, :] = jnp.swapaxes(ref[0], 0, 1)


def kernel(preds, regs):
    pin = jnp.transpose(preds, (1, 2, 0, 3))
    rin = jnp.transpose(regs, (1, 2, 0, 3))

    in_block = (1, _HH, _BS, _FW)
    p5, r5 = pl.pallas_call(
        _body,
        grid=(_NA, 2),
        in_specs=(
            [pl.BlockSpec(in_block, lambda a, ht, c=c: (c * _NA + a, ht, 0, 0))
             for c in range(2)]
            + [pl.BlockSpec(in_block, lambda a, ht, c=c: (c * _NA + a, ht, 0, 0))
               for c in range(4)]
        ),
        out_specs=[
            pl.BlockSpec((_BS, 1, _HH, 2, _FW), lambda a, ht: (0, a, ht, 0, 0)),
            pl.BlockSpec((_BS, 1, _HH, 4, _FW), lambda a, ht: (0, a, ht, 0, 0)),
        ],
        out_shape=[
            jax.ShapeDtypeStruct((_BS, _NA, _FH, 2, _FW), jnp.float32),
            jax.ShapeDtypeStruct((_BS, _NA, _FH, 4, _FW), jnp.float32),
        ],
    )(pin, pin, rin, rin, rin, rin)

    return (jnp.swapaxes(p5, 3, 4), jnp.swapaxes(r5, 3, 4))
